# baseline (speedup 1.0000x reference)
import numpy as np
import jax
import jax.numpy as jnp
from jax.experimental import pallas as pl
from jax.experimental.pallas import tpu as pltpu

_B, _L, _D = 1024, 64, 32
_K, _CBD = 32, 256
_S0 = 8
_BETA = 0.25
_EPS = 1e-3
_BM = 128
_F32 = jnp.float32

_pcall = pl.pallas_call
_CP = pltpu.CompilerParams(dimension_semantics=("arbitrary",))


def _band_conv(w, s_in):
    s_out = s_in - 1
    cin, cout = w.shape[2], w.shape[3]
    bands = np.zeros((2, s_in, s_out), np.float32)
    for jp in range(s_out):
        for dj in range(2):
            bands[dj, jp + dj, jp] = 1.0
    bands = jnp.asarray(bands)
    return [jnp.einsum('djs,dco->jcso', bands, w[di]).reshape(
        s_in * cin, s_out * cout) for di in range(2)]


def _band_convt(w, s_in):
    s_out = s_in + 1
    cin, cout = w.shape[2], w.shape[3]
    bands = np.zeros((2, s_in, s_out), np.float32)
    for jp in range(s_out):
        for dj in range(2):
            j = jp - 1 + dj
            if 0 <= j < s_in:
                bands[dj, j, jp] = 1.0
    bands = jnp.asarray(bands)
    return [jnp.einsum('djs,dco->jcso', bands, w[di]).reshape(
        s_in * cin, s_out * cout) for di in range(2)]


def _bn_affine(st, s_out, c, gamma, beta):
    n = float(_B * s_out * s_out)
    ssum = jnp.sum(st[0].reshape(s_out, c), axis=0)
    sq = jnp.sum(st[1].reshape(s_out, c), axis=0)
    mu = ssum / n
    var = sq / n - mu * mu
    sc = gamma / jnp.sqrt(var + _EPS)
    sh = beta - mu * sc
    return jnp.tile(sc, s_out)[None, :], jnp.tile(sh, s_out)[None, :]


def _stats_rows(vals, c):
    s = jnp.zeros((1, c), _F32)
    q = jnp.zeros((1, c), _F32)
    for v in vals:
        s = s + jnp.sum(v, axis=0, keepdims=True)
        q = q + jnp.sum(v * v, axis=0, keepdims=True)
    return jnp.concatenate([s, q, jnp.zeros((6, c), _F32)], axis=0)


def _scalar_pad(v):
    r = jax.lax.broadcasted_iota(jnp.int32, (8, 128), 0)
    c = jax.lax.broadcasted_iota(jnp.int32, (8, 128), 1)
    return jnp.where((r == 0) & (c == 0), v, 0.0)


def _dot(a, b):
    return jnp.dot(a, b, preferred_element_type=_F32)


def _convt_row(a, s_in, cin, ip, w0, w1):
    pieces = []
    for jp in range(s_in + 1):
        acc = None
        for di in range(2):
            k = ip - 1 + di
            if not 0 <= k < s_in:
                continue
            w = (w0, w1)[di]
            for dj in range(2):
                j = jp - 1 + dj
                if not 0 <= j < s_in:
                    continue
                term = _dot(a[k][:, j * cin:(j + 1) * cin],
                            w[dj * cin:(dj + 1) * cin, :])
                acc = term if acc is None else acc + term
        pieces.append(acc)
    return pieces


def _enc1_body(x_ref, w4_ref, y_ref, st_ref):
    @pl.when(pl.program_id(0) == 0)
    def _():
        st_ref[...] = jnp.zeros_like(st_ref)

    w4 = w4_ref[...]
    rows = []
    for ip in range(7):
        r0, r1 = x_ref[:, ip, :], x_ref[:, ip + 1, :]
        pieces = []
        for jp in range(7):
            sl = slice(jp * 32, jp * 32 + 64)
            pieces.append(_dot(
                jnp.concatenate([r0[:, sl], r1[:, sl]], axis=1), w4))
        y = jnp.concatenate(pieces, axis=1)
        y_ref[:, ip, :] = y
        rows.append(y)
    st_ref[...] += _stats_rows(rows, 7 * 64)


def _enc2_body(y1_ref, sc_ref, sh_ref, w4_ref, y_ref, st_ref):
    @pl.when(pl.program_id(0) == 0)
    def _():
        st_ref[...] = jnp.zeros_like(st_ref)

    sc, sh = sc_ref[...], sh_ref[...]
    a = [jnp.maximum(y1_ref[:, k, :] * sc + sh, 0.0) for k in range(7)]
    w4 = w4_ref[...]
    rows = []
    for ip in range(6):
        pieces = []
        for jp in range(6):
            sl = slice(jp * 64, jp * 64 + 128)
            pieces.append(_dot(
                jnp.concatenate([a[ip][:, sl], a[ip + 1][:, sl]], axis=1),
                w4))
        y = jnp.concatenate(pieces, axis=1)
        y_ref[:, ip, :] = y
        rows.append(y)
    st_ref[...] += _stats_rows(rows, 6 * 128)


def _vq_body(y2_ref, sc_ref, sh_ref, w4_ref, b3_ref, cbt_ref,
             cbn_ref, cb_ref, t0_ref, t1_ref,
             ze_ref, zq_ref, g1_ref, st_ref, vq_ref):
    @pl.when(pl.program_id(0) == 0)
    def _():
        st_ref[...] = jnp.zeros_like(st_ref)
        vq_ref[...] = jnp.zeros_like(vq_ref)

    sc, sh = sc_ref[...], sh_ref[...]
    a = [jnp.maximum(y2_ref[:, k, :] * sc + sh, 0.0) for k in range(6)]
    w4 = w4_ref[...]
    b3 = b3_ref[...]
    cbt, cbn, cb = cbt_ref[...], cbn_ref[...], cb_ref[...]

    zq_p = []
    vq_acc = jnp.float32(0.0)
    for ip in range(5):
        ze_pieces = []
        for jp in range(5):
            sl = slice(jp * 128, jp * 128 + 256)
            z = (_dot(jnp.concatenate([a[ip][:, sl], a[ip + 1][:, sl]],
                                      axis=1), w4)
                 + b3[:, jp * 256:(jp + 1) * 256])
            ze_pieces.append(z)
        ze_row = jnp.concatenate(ze_pieces, axis=1)
        ze_ref[:, ip, :] = ze_row
        zq_pieces = []
        for jp in range(5):
            zej = ze_pieces[jp]
            d = (jnp.sum(zej * zej, axis=1, keepdims=True)
                 - 2.0 * _dot(zej, cbt) + cbn)
            mn = jnp.min(d, axis=1, keepdims=True)
            iota = jax.lax.broadcasted_iota(jnp.int32, d.shape, 1)
            big = jnp.where(d == mn, iota, _K)
            jmin = jnp.min(big, axis=1, keepdims=True)
            zq_pieces.append(_dot((iota == jmin).astype(_F32), cb))
        zq_row = jnp.concatenate(zq_pieces, axis=1)
        zq_ref[:, ip, :] = zq_row
        zq_p.append(zq_pieces)
        diff = zq_row - ze_row
        vq_acc = vq_acc + jnp.sum(diff * diff)
    vq_ref[...] += _scalar_pad(vq_acc)

    zq_b = [[z.astype(jnp.bfloat16) for z in row] for row in zq_p]
    t0, t1 = t0_ref[...], t1_ref[...]
    rows = []
    for ip in range(6):
        pieces = []
        for jp in range(6):
            acc = None
            for di in range(2):
                k = ip - 1 + di
                if not 0 <= k < 5:
                    continue
                t = (t0, t1)[di]
                for dj in range(2):
                    j = jp - 1 + dj
                    if not 0 <= j < 5:
                        continue
                    term = _dot(zq_b[k][j], t[dj * 256:(dj + 1) * 256, :])
                    acc = term if acc is None else acc + term
            pieces.append(acc)
        g = jnp.concatenate(pieces, axis=1)
        g1_ref[:, ip, :] = g.astype(jnp.bfloat16)
        rows.append(g)
    st_ref[...] += _stats_rows(rows, 6 * 128)


def _dec2_body(g1_ref, sc_ref, sh_ref, t0_ref, t1_ref, y_ref, st_ref):
    @pl.when(pl.program_id(0) == 0)
    def _():
        st_ref[...] = jnp.zeros_like(st_ref)

    sc, sh = sc_ref[...], sh_ref[...]
    a = [jnp.maximum(g1_ref[:, k, :].astype(_F32) * sc + sh,
                     0.0).astype(jnp.bfloat16) for k in range(6)]
    t0, t1 = t0_ref[...], t1_ref[...]
    rows = []
    for ip in range(7):
        pieces = _convt_row(a, 6, 128, ip, t0, t1)
        y = jnp.concatenate(pieces, axis=1)
        y_ref[:, ip, :] = y.astype(jnp.bfloat16)
        rows.append(y)
    st_ref[...] += _stats_rows(rows, 7 * 64)


def _dec3_body(g2_ref, sc_ref, sh_ref, t0_ref, t1_ref, b3_ref, x_ref,
               mask_ref, vx_ref, vm_ref, rec_ref):
    @pl.when(pl.program_id(0) == 0)
    def _():
        rec_ref[...] = jnp.zeros_like(rec_ref)

    sc, sh = sc_ref[...], sh_ref[...]
    a = [jnp.maximum(g2_ref[:, k, :].astype(_F32) * sc + sh,
                     0.0).astype(jnp.bfloat16) for k in range(7)]
    t0, t1 = t0_ref[...], t1_ref[...]
    b3 = b3_ref[...]
    rec_acc = jnp.float32(0.0)
    rowsum = jnp.zeros((vx_ref.shape[0], 256), _F32)
    for ip in range(8):
        g = None
        for di in range(2):
            k = ip - 1 + di
            if 0 <= k < 7:
                term = _dot(a[k], t0 if di == 0 else t1)
                g = term if g is None else g + term
        g = g + b3
        vx_ref[:, ip, :] = g
        rowsum = rowsum + g
        d = x_ref[:, ip, :] - g
        rec_acc = rec_acc + jnp.sum(d * d)
    rec_ref[...] += _scalar_pad(rec_acc)
    vsum = jnp.zeros((vx_ref.shape[0], 32), _F32)
    for j in range(8):
        vsum = vsum + rowsum[:, j * 32:(j + 1) * 32]
    msum = jnp.sum(mask_ref[...], axis=1, keepdims=True)
    vm_ref[...] = vsum / msum


def _full(shape):
    nd = len(shape)
    return pl.BlockSpec(shape, lambda i: (0,) * nd)


def _btile(shape):
    nd = len(shape)
    return pl.BlockSpec((_BM,) + shape[1:], lambda i: (i,) + (0,) * (nd - 1))


def kernel(x, mask, code_book, params):
    p = params
    grid = (_B // _BM,)

    xf = jnp.reshape(x, (_B, _S0, _S0 * _D)).astype(_F32)
    mask = mask.astype(_F32)
    cb = code_book.astype(_F32)

    w1 = p['ew1'].reshape(128, 64)
    w2 = p['ew2'].reshape(256, 128)
    w3 = p['ew3'].reshape(512, 256)
    bf16 = jnp.bfloat16
    t1 = [p['dw1'][di].reshape(512, 128).astype(bf16) for di in range(2)]
    t2 = [p['dw2'][di].reshape(256, 64).astype(bf16) for di in range(2)]
    t3 = [m.astype(bf16) for m in _band_convt(p['dw3'], 7)]
    b3 = jnp.tile(p['eb3'], 5)[None, :]
    db3 = jnp.tile(p['db3'], 8)[None, :]
    cbt = cb.T
    cbn = jnp.sum(cb * cb, axis=1)[None, :]

    y1, st1 = _pcall(
        _enc1_body, grid=grid,
        in_specs=[_btile((_B, 8, 256)), _full((128, 64))],
        out_specs=[_btile((_B, 7, 448)), _full((8, 448))],
        out_shape=[jax.ShapeDtypeStruct((_B, 7, 448), _F32),
                   jax.ShapeDtypeStruct((8, 448), _F32)],
        compiler_params=_CP,
    )(xf, w1)
    sc1, sh1 = _bn_affine(st1, 7, 64, p['eg1'], p['ebe1'])

    y2, st2 = _pcall(
        _enc2_body, grid=grid,
        in_specs=[_btile((_B, 7, 448)), _full((1, 448)), _full((1, 448)),
                  _full((256, 128))],
        out_specs=[_btile((_B, 6, 768)), _full((8, 768))],
        out_shape=[jax.ShapeDtypeStruct((_B, 6, 768), _F32),
                   jax.ShapeDtypeStruct((8, 768), _F32)],
        compiler_params=_CP,
    )(y1, sc1, sh1, w2)
    sc2, sh2 = _bn_affine(st2, 6, 128, p['eg2'], p['ebe2'])

    ze, zq, g1, st3, vqs = _pcall(
        _vq_body, grid=grid,
        in_specs=[_btile((_B, 6, 768)), _full((1, 768)), _full((1, 768)),
                  _full((512, 256)), _full((1, 1280)),
                  _full((256, 32)), _full((1, 32)), _full((32, 256)),
                  _full((512, 128)), _full((512, 128))],
        out_specs=[_btile((_B, 5, 1280)), _btile((_B, 5, 1280)),
                   _btile((_B, 6, 768)), _full((8, 768)), _full((8, 128))],
        out_shape=[jax.ShapeDtypeStruct((_B, 5, 1280), _F32),
                   jax.ShapeDtypeStruct((_B, 5, 1280), _F32),
                   jax.ShapeDtypeStruct((_B, 6, 768), jnp.bfloat16),
                   jax.ShapeDtypeStruct((8, 768), _F32),
                   jax.ShapeDtypeStruct((8, 128), _F32)],
        compiler_params=_CP,
    )(y2, sc2, sh2, w3, b3, cbt, cbn, cb, t1[0], t1[1])
    sc3, sh3 = _bn_affine(st3, 6, 128, p['dg1'], p['dbe1'])

    g2, st4 = _pcall(
        _dec2_body, grid=grid,
        in_specs=[_btile((_B, 6, 768)), _full((1, 768)), _full((1, 768)),
                  _full((256, 64)), _full((256, 64))],
        out_specs=[_btile((_B, 7, 448)), _full((8, 448))],
        out_shape=[jax.ShapeDtypeStruct((_B, 7, 448), jnp.bfloat16),
                   jax.ShapeDtypeStruct((8, 448), _F32)],
        compiler_params=_CP,
    )(g1, sc3, sh3, t2[0], t2[1])
    sc4, sh4 = _bn_affine(st4, 7, 64, p['dg2'], p['dbe2'])

    vx, vm, rec = _pcall(
        _dec3_body, grid=grid,
        in_specs=[_btile((_B, 7, 448)), _full((1, 448)), _full((1, 448)),
                  _full((448, 256)), _full((448, 256)), _full((1, 256)),
                  _btile((_B, 8, 256)), _btile((_B, 64))],
        out_specs=[_btile((_B, 8, 256)), _btile((_B, 32)), _full((8, 128))],
        out_shape=[jax.ShapeDtypeStruct((_B, 8, 256), _F32),
                   jax.ShapeDtypeStruct((_B, 32), _F32),
                   jax.ShapeDtypeStruct((8, 128), _F32)],
        compiler_params=_CP,
    )(g2, sc4, sh4, t3[0], t3[1], db3, xf, mask)

    ze_out = jnp.reshape(ze, (_B * 25, _CBD))
    zq_out = jnp.reshape(zq, (_B * 25, _CBD))
    vq_x = jnp.reshape(vx, (_B, _L, _D))
    recon = rec[0, 0] / float(_B * _L * _D)
    vq_term = vqs[0, 0] / float(_B * 25 * _CBD)
    loss = recon + vq_term + _BETA * vq_term
    return (vm, vq_x, ze_out, zq_out, loss)


# speedup vs baseline: 1.1216x; 1.1216x over previous
import numpy as np
import jax
import jax.numpy as jnp
from jax.experimental import pallas as pl
from jax.experimental.pallas import tpu as pltpu

_B, _L, _D = 1024, 64, 32
_K, _CBD = 32, 256
_S0 = 8
_BETA = 0.25
_EPS = 1e-3
_BM = 128
_F32 = jnp.float32

_pcall = pl.pallas_call
_CP = pltpu.CompilerParams(dimension_semantics=("arbitrary",))


def _band_conv(w, s_in):
    s_out = s_in - 1
    cin, cout = w.shape[2], w.shape[3]
    bands = np.zeros((2, s_in, s_out), np.float32)
    for jp in range(s_out):
        for dj in range(2):
            bands[dj, jp + dj, jp] = 1.0
    bands = jnp.asarray(bands)
    return [jnp.einsum('djs,dco->jcso', bands, w[di]).reshape(
        s_in * cin, s_out * cout) for di in range(2)]


def _band_convt(w, s_in):
    s_out = s_in + 1
    cin, cout = w.shape[2], w.shape[3]
    bands = np.zeros((2, s_in, s_out), np.float32)
    for jp in range(s_out):
        for dj in range(2):
            j = jp - 1 + dj
            if 0 <= j < s_in:
                bands[dj, j, jp] = 1.0
    bands = jnp.asarray(bands)
    return [jnp.einsum('djs,dco->jcso', bands, w[di]).reshape(
        s_in * cin, s_out * cout) for di in range(2)]


def _affine_from_stats(st_ref, s, c, g_ref, b_ref, n):
    row0 = st_ref[0:1, :]
    row1 = st_ref[1:2, :]
    ssum = jnp.zeros((1, c), _F32)
    sq = jnp.zeros((1, c), _F32)
    for j in range(s):
        ssum = ssum + row0[:, j * c:(j + 1) * c]
        sq = sq + row1[:, j * c:(j + 1) * c]
    mu = ssum / n
    var = sq / n - mu * mu
    scv = g_ref[...] / jnp.sqrt(var + _EPS)
    shv = b_ref[...] - mu * scv
    return (jnp.concatenate([scv] * s, axis=1),
            jnp.concatenate([shv] * s, axis=1))


def _stats_rows(vals, c):
    s = jnp.zeros((1, c), _F32)
    q = jnp.zeros((1, c), _F32)
    for v in vals:
        s = s + jnp.sum(v, axis=0, keepdims=True)
        q = q + jnp.sum(v * v, axis=0, keepdims=True)
    return jnp.concatenate([s, q, jnp.zeros((6, c), _F32)], axis=0)


def _scalar_pad(v):
    r = jax.lax.broadcasted_iota(jnp.int32, (8, 128), 0)
    c = jax.lax.broadcasted_iota(jnp.int32, (8, 128), 1)
    return jnp.where((r == 0) & (c == 0), v, 0.0)


def _dot(a, b):
    return jnp.dot(a, b, preferred_element_type=_F32)


def _convt_row(a, s_in, cin, ip, w0, w1):
    pieces = []
    for jp in range(s_in + 1):
        acc = None
        for di in range(2):
            k = ip - 1 + di
            if not 0 <= k < s_in:
                continue
            w = (w0, w1)[di]
            for dj in range(2):
                j = jp - 1 + dj
                if not 0 <= j < s_in:
                    continue
                term = _dot(a[k][:, j * cin:(j + 1) * cin],
                            w[dj * cin:(dj + 1) * cin, :])
                acc = term if acc is None else acc + term
        pieces.append(acc)
    return pieces


def _enc1_body(x_ref, w4_ref, y_ref, st_ref):
    @pl.when(pl.program_id(0) == 0)
    def _():
        st_ref[...] = jnp.zeros_like(st_ref)

    w4 = w4_ref[...]
    rows = []
    for ip in range(7):
        r0, r1 = x_ref[:, ip, :], x_ref[:, ip + 1, :]
        pieces = []
        for jp in range(7):
            sl = slice(jp * 32, jp * 32 + 64)
            pieces.append(_dot(
                jnp.concatenate([r0[:, sl], r1[:, sl]], axis=1), w4))
        y = jnp.concatenate(pieces, axis=1)
        y_ref[:, ip, :] = y
        rows.append(y)
    st_ref[...] += _stats_rows(rows, 7 * 64)


def _enc2_body(y1_ref, st_in_ref, g_ref, b_ref, w4_ref, y_ref, st_ref):
    @pl.when(pl.program_id(0) == 0)
    def _():
        st_ref[...] = jnp.zeros_like(st_ref)

    sc, sh = _affine_from_stats(st_in_ref, 7, 64, g_ref, b_ref,
                                float(_B * 49))
    a = [jnp.maximum(y1_ref[:, k, :] * sc + sh, 0.0) for k in range(7)]
    w4 = w4_ref[...]
    rows = []
    for ip in range(6):
        pieces = []
        for jp in range(6):
            sl = slice(jp * 64, jp * 64 + 128)
            pieces.append(_dot(
                jnp.concatenate([a[ip][:, sl], a[ip + 1][:, sl]], axis=1),
                w4))
        y = jnp.concatenate(pieces, axis=1)
        y_ref[:, ip, :] = y
        rows.append(y)
    st_ref[...] += _stats_rows(rows, 6 * 128)


def _vq_body(y2_ref, st_in_ref, g_ref, b_ref, w4_ref, b3_ref, cbt_ref,
             cbn_ref, cb_ref, t0_ref, t1_ref,
             ze_ref, zq_ref, g1_ref, st_ref, vq_ref):
    @pl.when(pl.program_id(0) == 0)
    def _():
        st_ref[...] = jnp.zeros_like(st_ref)
        vq_ref[...] = jnp.zeros_like(vq_ref)

    sc, sh = _affine_from_stats(st_in_ref, 6, 128, g_ref, b_ref,
                                float(_B * 36))
    a = [jnp.maximum(y2_ref[:, k, :] * sc + sh, 0.0) for k in range(6)]
    w4 = w4_ref[...]
    b3 = b3_ref[...]
    cbt, cbn, cb = cbt_ref[...], cbn_ref[...], cb_ref[...]

    zq_p = []
    vq_acc = jnp.float32(0.0)
    for ip in range(5):
        ze_pieces = []
        for jp in range(5):
            sl = slice(jp * 128, jp * 128 + 256)
            z = (_dot(jnp.concatenate([a[ip][:, sl], a[ip + 1][:, sl]],
                                      axis=1), w4)
                 + b3[:, jp * 256:(jp + 1) * 256])
            ze_pieces.append(z)
        ze_row = jnp.concatenate(ze_pieces, axis=1)
        ze_ref[:, ip, :] = ze_row
        zq_pieces = []
        for jp in range(5):
            zej = ze_pieces[jp]
            d = (jnp.sum(zej * zej, axis=1, keepdims=True)
                 - 2.0 * _dot(zej, cbt) + cbn)
            mn = jnp.min(d, axis=1, keepdims=True)
            iota = jax.lax.broadcasted_iota(jnp.int32, d.shape, 1)
            big = jnp.where(d == mn, iota, _K)
            jmin = jnp.min(big, axis=1, keepdims=True)
            zq_pieces.append(_dot((iota == jmin).astype(_F32), cb))
        zq_row = jnp.concatenate(zq_pieces, axis=1)
        zq_ref[:, ip, :] = zq_row
        zq_p.append(zq_pieces)
        diff = zq_row - ze_row
        vq_acc = vq_acc + jnp.sum(diff * diff)
    vq_ref[...] += _scalar_pad(vq_acc)

    t0, t1 = t0_ref[...], t1_ref[...]
    rows = []
    for ip in range(6):
        pieces = []
        for jp in range(6):
            acc = None
            for di in range(2):
                k = ip - 1 + di
                if not 0 <= k < 5:
                    continue
                t = (t0, t1)[di]
                for dj in range(2):
                    j = jp - 1 + dj
                    if not 0 <= j < 5:
                        continue
                    term = _dot(zq_p[k][j], t[dj * 256:(dj + 1) * 256, :])
                    acc = term if acc is None else acc + term
            pieces.append(acc)
        g = jnp.concatenate(pieces, axis=1)
        g1_ref[:, ip, :] = g
        rows.append(g)
    st_ref[...] += _stats_rows(rows, 6 * 128)


def _dec2_body(g1_ref, st_in_ref, g_ref, b_ref, t0_ref, t1_ref, y_ref,
               st_ref):
    @pl.when(pl.program_id(0) == 0)
    def _():
        st_ref[...] = jnp.zeros_like(st_ref)

    sc, sh = _affine_from_stats(st_in_ref, 6, 128, g_ref, b_ref,
                                float(_B * 36))
    a = [jnp.maximum(g1_ref[:, k, :] * sc + sh, 0.0) for k in range(6)]
    t0, t1 = t0_ref[...], t1_ref[...]
    rows = []
    for ip in range(7):
        pieces = _convt_row(a, 6, 128, ip, t0, t1)
        y = jnp.concatenate(pieces, axis=1)
        y_ref[:, ip, :] = y
        rows.append(y)
    st_ref[...] += _stats_rows(rows, 7 * 64)


def _dec3_body(g2_ref, st_in_ref, g_ref, b_ref, t0_ref, t1_ref, b3_ref,
               x_ref, mask_ref, vx_ref, vm_ref, rec_ref):
    @pl.when(pl.program_id(0) == 0)
    def _():
        rec_ref[...] = jnp.zeros_like(rec_ref)

    sc, sh = _affine_from_stats(st_in_ref, 7, 64, g_ref, b_ref,
                                float(_B * 49))
    a = [jnp.maximum(g2_ref[:, k, :] * sc + sh, 0.0) for k in range(7)]
    t0, t1 = t0_ref[...], t1_ref[...]
    b3 = b3_ref[...]
    rec_acc = jnp.float32(0.0)
    rowsum = jnp.zeros((vx_ref.shape[0], 256), _F32)
    for ip in range(8):
        g = None
        for di in range(2):
            k = ip - 1 + di
            if 0 <= k < 7:
                term = _dot(a[k], t0 if di == 0 else t1)
                g = term if g is None else g + term
        g = g + b3
        vx_ref[:, ip, :] = g
        rowsum = rowsum + g
        d = x_ref[:, ip, :] - g
        rec_acc = rec_acc + jnp.sum(d * d)
    rec_ref[...] += _scalar_pad(rec_acc)
    vsum = jnp.zeros((vx_ref.shape[0], 32), _F32)
    for j in range(8):
        vsum = vsum + rowsum[:, j * 32:(j + 1) * 32]
    msum = jnp.sum(mask_ref[...], axis=1, keepdims=True)
    vm_ref[...] = vsum / msum


def _full(shape):
    nd = len(shape)
    return pl.BlockSpec(shape, lambda i: (0,) * nd)


def _btile(shape):
    nd = len(shape)
    return pl.BlockSpec((_BM,) + shape[1:], lambda i: (i,) + (0,) * (nd - 1))


def kernel(x, mask, code_book, params):
    p = params
    grid = (_B // _BM,)

    xf = jnp.reshape(x, (_B, _S0, _S0 * _D)).astype(_F32)
    mask = mask.astype(_F32)
    cb = code_book.astype(_F32)

    w1 = p['ew1'].reshape(128, 64)
    w2 = p['ew2'].reshape(256, 128)
    w3 = p['ew3'].reshape(512, 256)
    t1 = [p['dw1'][di].reshape(512, 128) for di in range(2)]
    t2 = [p['dw2'][di].reshape(256, 64) for di in range(2)]
    t3 = _band_convt(p['dw3'], 7)
    b3 = jnp.tile(p['eb3'], 5)[None, :]
    db3 = jnp.tile(p['db3'], 8)[None, :]
    cbt = cb.T
    cbn = jnp.sum(cb * cb, axis=1)[None, :]

    y1, st1 = _pcall(
        _enc1_body, grid=grid,
        in_specs=[_btile((_B, 8, 256)), _full((128, 64))],
        out_specs=[_btile((_B, 7, 448)), _full((8, 448))],
        out_shape=[jax.ShapeDtypeStruct((_B, 7, 448), _F32),
                   jax.ShapeDtypeStruct((8, 448), _F32)],
        compiler_params=_CP,
    )(xf, w1)
    eg1, ebe1 = p['eg1'][None, :], p['ebe1'][None, :]

    y2, st2 = _pcall(
        _enc2_body, grid=grid,
        in_specs=[_btile((_B, 7, 448)), _full((8, 448)), _full((1, 64)),
                  _full((1, 64)), _full((256, 128))],
        out_specs=[_btile((_B, 6, 768)), _full((8, 768))],
        out_shape=[jax.ShapeDtypeStruct((_B, 6, 768), _F32),
                   jax.ShapeDtypeStruct((8, 768), _F32)],
        compiler_params=_CP,
    )(y1, st1, eg1, ebe1, w2)

    ze, zq, g1, st3, vqs = _pcall(
        _vq_body, grid=grid,
        in_specs=[_btile((_B, 6, 768)), _full((8, 768)), _full((1, 128)),
                  _full((1, 128)), _full((512, 256)), _full((1, 1280)),
                  _full((256, 32)), _full((1, 32)), _full((32, 256)),
                  _full((512, 128)), _full((512, 128))],
        out_specs=[_btile((_B, 5, 1280)), _btile((_B, 5, 1280)),
                   _btile((_B, 6, 768)), _full((8, 768)), _full((8, 128))],
        out_shape=[jax.ShapeDtypeStruct((_B, 5, 1280), _F32),
                   jax.ShapeDtypeStruct((_B, 5, 1280), _F32),
                   jax.ShapeDtypeStruct((_B, 6, 768), _F32),
                   jax.ShapeDtypeStruct((8, 768), _F32),
                   jax.ShapeDtypeStruct((8, 128), _F32)],
        compiler_params=_CP,
    )(y2, st2, p['eg2'][None, :], p['ebe2'][None, :], w3, b3, cbt, cbn,
      cb, t1[0], t1[1])

    g2, st4 = _pcall(
        _dec2_body, grid=grid,
        in_specs=[_btile((_B, 6, 768)), _full((8, 768)), _full((1, 128)),
                  _full((1, 128)), _full((256, 64)), _full((256, 64))],
        out_specs=[_btile((_B, 7, 448)), _full((8, 448))],
        out_shape=[jax.ShapeDtypeStruct((_B, 7, 448), _F32),
                   jax.ShapeDtypeStruct((8, 448), _F32)],
        compiler_params=_CP,
    )(g1, st3, p['dg1'][None, :], p['dbe1'][None, :], t2[0], t2[1])

    vx, vm, rec = _pcall(
        _dec3_body, grid=grid,
        in_specs=[_btile((_B, 7, 448)), _full((8, 448)), _full((1, 64)),
                  _full((1, 64)), _full((448, 256)), _full((448, 256)),
                  _full((1, 256)), _btile((_B, 8, 256)), _btile((_B, 64))],
        out_specs=[_btile((_B, 8, 256)), _btile((_B, 32)), _full((8, 128))],
        out_shape=[jax.ShapeDtypeStruct((_B, 8, 256), _F32),
                   jax.ShapeDtypeStruct((_B, 32), _F32),
                   jax.ShapeDtypeStruct((8, 128), _F32)],
        compiler_params=_CP,
    )(g2, st4, p['dg2'][None, :], p['dbe2'][None, :], t3[0], t3[1], db3,
      xf, mask)

    ze_out = jnp.reshape(ze, (_B * 25, _CBD))
    zq_out = jnp.reshape(zq, (_B * 25, _CBD))
    vq_x = jnp.reshape(vx, (_B, _L, _D))
    recon = rec[0, 0] / float(_B * _L * _D)
    vq_term = vqs[0, 0] / float(_B * 25 * _CBD)
    loss = recon + vq_term + _BETA * vq_term
    return (vm, vq_x, ze_out, zq_out, loss)


# speedup vs baseline: 1.1350x; 1.0119x over previous
import numpy as np
import jax
import jax.numpy as jnp
from jax.experimental import pallas as pl
from jax.experimental.pallas import tpu as pltpu

_B, _L, _D = 1024, 64, 32
_K, _CBD = 32, 256
_S0 = 8
_BETA = 0.25
_EPS = 1e-3
_BM = 128
_F32 = jnp.float32

_pcall = pl.pallas_call
_CP = pltpu.CompilerParams(dimension_semantics=("arbitrary",))


def _band_conv(w, s_in):
    s_out = s_in - 1
    cin, cout = w.shape[2], w.shape[3]
    bands = np.zeros((2, s_in, s_out), np.float32)
    for jp in range(s_out):
        for dj in range(2):
            bands[dj, jp + dj, jp] = 1.0
    bands = jnp.asarray(bands)
    return [jnp.einsum('djs,dco->jcso', bands, w[di]).reshape(
        s_in * cin, s_out * cout) for di in range(2)]


def _band_convt(w, s_in):
    s_out = s_in + 1
    cin, cout = w.shape[2], w.shape[3]
    bands = np.zeros((2, s_in, s_out), np.float32)
    for jp in range(s_out):
        for dj in range(2):
            j = jp - 1 + dj
            if 0 <= j < s_in:
                bands[dj, j, jp] = 1.0
    bands = jnp.asarray(bands)
    return [jnp.einsum('djs,dco->jcso', bands, w[di]).reshape(
        s_in * cin, s_out * cout) for di in range(2)]


def _affine_from_stats(st_ref, s, c, g_ref, b_ref, n):
    row0 = st_ref[0:1, :]
    row1 = st_ref[1:2, :]
    ssum = jnp.zeros((1, c), _F32)
    sq = jnp.zeros((1, c), _F32)
    for j in range(s):
        ssum = ssum + row0[:, j * c:(j + 1) * c]
        sq = sq + row1[:, j * c:(j + 1) * c]
    mu = ssum / n
    var = sq / n - mu * mu
    scv = g_ref[...] / jnp.sqrt(var + _EPS)
    shv = b_ref[...] - mu * scv
    return (jnp.concatenate([scv] * s, axis=1),
            jnp.concatenate([shv] * s, axis=1))


def _stats_rows(vals, c):
    s = jnp.zeros((1, c), _F32)
    q = jnp.zeros((1, c), _F32)
    for v in vals:
        s = s + jnp.sum(v, axis=0, keepdims=True)
        q = q + jnp.sum(v * v, axis=0, keepdims=True)
    return jnp.concatenate([s, q, jnp.zeros((6, c), _F32)], axis=0)


def _scalar_pad(v):
    r = jax.lax.broadcasted_iota(jnp.int32, (8, 128), 0)
    c = jax.lax.broadcasted_iota(jnp.int32, (8, 128), 1)
    return jnp.where((r == 0) & (c == 0), v, 0.0)


def _dot(a, b):
    return jnp.dot(a, b, preferred_element_type=_F32)


def _convt_row(a, s_in, cin, ip, w0, w1):
    pieces = []
    for jp in range(s_in + 1):
        acc = None
        for di in range(2):
            k = ip - 1 + di
            if not 0 <= k < s_in:
                continue
            w = (w0, w1)[di]
            for dj in range(2):
                j = jp - 1 + dj
                if not 0 <= j < s_in:
                    continue
                term = _dot(a[k][:, j * cin:(j + 1) * cin],
                            w[dj * cin:(dj + 1) * cin, :])
                acc = term if acc is None else acc + term
        pieces.append(acc)
    return pieces


def _enc1_body(x_ref, w4_ref, y_ref, st_ref):
    @pl.when(pl.program_id(0) == 0)
    def _():
        st_ref[...] = jnp.zeros_like(st_ref)

    w4 = w4_ref[...]
    rows = []
    for ip in range(7):
        r0, r1 = x_ref[:, ip, :], x_ref[:, ip + 1, :]
        pieces = []
        for jp in range(7):
            sl = slice(jp * 32, jp * 32 + 64)
            pieces.append(_dot(
                jnp.concatenate([r0[:, sl], r1[:, sl]], axis=1), w4))
        y = jnp.concatenate(pieces, axis=1)
        y_ref[:, ip, :] = y
        rows.append(y)
    st_ref[...] += _stats_rows(rows, 7 * 64)


def _enc2_body(y1_ref, st_in_ref, g_ref, b_ref, w4_ref, y_ref, st_ref):
    @pl.when(pl.program_id(0) == 0)
    def _():
        st_ref[...] = jnp.zeros_like(st_ref)

    sc, sh = _affine_from_stats(st_in_ref, 7, 64, g_ref, b_ref,
                                float(_B * 49))
    a = [jnp.maximum(y1_ref[:, k, :] * sc + sh, 0.0) for k in range(7)]
    w4 = w4_ref[...]
    rows = []
    for ip in range(6):
        pieces = []
        for jp in range(6):
            sl = slice(jp * 64, jp * 64 + 128)
            pieces.append(_dot(
                jnp.concatenate([a[ip][:, sl], a[ip + 1][:, sl]], axis=1),
                w4))
        y = jnp.concatenate(pieces, axis=1)
        y_ref[:, ip, :] = y
        rows.append(y)
    st_ref[...] += _stats_rows(rows, 6 * 128)


def _vq_body(y2_ref, st_in_ref, g_ref, b_ref, w4_ref, b3_ref, cbt_ref,
             cbn_ref, cb_ref, t0_ref, t1_ref,
             ze_ref, zq_ref, g1_ref, st_ref, vq_ref):
    @pl.when(pl.program_id(0) == 0)
    def _():
        st_ref[...] = jnp.zeros_like(st_ref)
        vq_ref[...] = jnp.zeros_like(vq_ref)

    sc, sh = _affine_from_stats(st_in_ref, 6, 128, g_ref, b_ref,
                                float(_B * 36))
    a = [jnp.maximum(y2_ref[:, k, :] * sc + sh, 0.0) for k in range(6)]
    w4 = w4_ref[...]
    b3 = b3_ref[...]
    cbt, cbn, cb = cbt_ref[...], cbn_ref[...], cb_ref[...]

    zq_p = []
    vq_acc = jnp.float32(0.0)
    for ip in range(5):
        ze_pieces = []
        for jp in range(5):
            sl = slice(jp * 128, jp * 128 + 256)
            z = (_dot(jnp.concatenate([a[ip][:, sl], a[ip + 1][:, sl]],
                                      axis=1), w4)
                 + b3[:, jp * 256:(jp + 1) * 256])
            ze_pieces.append(z)
        ze_row = jnp.concatenate(ze_pieces, axis=1)
        ze_ref[:, ip, :] = ze_row
        zq_pieces = []
        for jp in range(5):
            zej = ze_pieces[jp]
            d = (jnp.sum(zej * zej, axis=1, keepdims=True)
                 - 2.0 * _dot(zej, cbt) + cbn)
            mn = jnp.min(d, axis=1, keepdims=True)
            iota = jax.lax.broadcasted_iota(jnp.int32, d.shape, 1)
            big = jnp.where(d == mn, iota, _K)
            jmin = jnp.min(big, axis=1, keepdims=True)
            zq_pieces.append(_dot((iota == jmin).astype(_F32), cb))
        zq_row = jnp.concatenate(zq_pieces, axis=1)
        zq_ref[:, ip, :] = zq_row
        zq_p.append(zq_pieces)
        diff = zq_row - ze_row
        vq_acc = vq_acc + jnp.sum(diff * diff)
    vq_ref[...] += _scalar_pad(vq_acc)

    t0, t1 = t0_ref[...], t1_ref[...]
    rows = []
    for ip in range(6):
        pieces = []
        for jp in range(6):
            acc = None
            for di in range(2):
                k = ip - 1 + di
                if not 0 <= k < 5:
                    continue
                t = (t0, t1)[di]
                for dj in range(2):
                    j = jp - 1 + dj
                    if not 0 <= j < 5:
                        continue
                    term = _dot(zq_p[k][j], t[dj * 256:(dj + 1) * 256, :])
                    acc = term if acc is None else acc + term
            pieces.append(acc)
        g = jnp.concatenate(pieces, axis=1)
        g1_ref[:, ip, :] = g
        rows.append(g)
    st_ref[...] += _stats_rows(rows, 6 * 128)


def _dec2_body(g1_ref, st_in_ref, g_ref, b_ref, t0_ref, t1_ref, y_ref,
               st_ref):
    @pl.when(pl.program_id(0) == 0)
    def _():
        st_ref[...] = jnp.zeros_like(st_ref)

    sc, sh = _affine_from_stats(st_in_ref, 6, 128, g_ref, b_ref,
                                float(_B * 36))
    a = [jnp.maximum(g1_ref[:, k, :] * sc + sh, 0.0) for k in range(6)]
    t0, t1 = t0_ref[...], t1_ref[...]
    rows = []
    for ip in range(7):
        pieces = _convt_row(a, 6, 128, ip, t0, t1)
        y = jnp.concatenate(pieces, axis=1)
        y_ref[:, ip, :] = y
        rows.append(y)
    st_ref[...] += _stats_rows(rows, 7 * 64)


def _dec3_body(g2_ref, st_in_ref, g_ref, b_ref, t0_ref, t1_ref, b3_ref,
               x_ref, mask_ref, vx_ref, vm_ref, rec_ref):
    @pl.when(pl.program_id(0) == 0)
    def _():
        rec_ref[...] = jnp.zeros_like(rec_ref)

    sc, sh = _affine_from_stats(st_in_ref, 7, 64, g_ref, b_ref,
                                float(_B * 49))
    a = [jnp.maximum(g2_ref[:, k, :] * sc + sh, 0.0) for k in range(7)]
    t0, t1 = t0_ref[...], t1_ref[...]
    b3 = b3_ref[...]
    rec_acc = jnp.float32(0.0)
    rowsum = jnp.zeros((vx_ref.shape[0], 256), _F32)
    for ip in range(8):
        g = None
        for di in range(2):
            k = ip - 1 + di
            if 0 <= k < 7:
                term = _dot(a[k], t0 if di == 0 else t1)
                g = term if g is None else g + term
        g = g + b3
        vx_ref[:, ip, :] = g
        rowsum = rowsum + g
        d = x_ref[:, ip, :] - g
        rec_acc = rec_acc + jnp.sum(d * d)
    rec_ref[...] += _scalar_pad(rec_acc)
    vsum = jnp.zeros((vx_ref.shape[0], 32), _F32)
    for j in range(8):
        vsum = vsum + rowsum[:, j * 32:(j + 1) * 32]
    msum = jnp.sum(mask_ref[...], axis=1, keepdims=True)
    vm_ref[...] = vsum / msum


def _full(shape):
    nd = len(shape)
    return pl.BlockSpec(shape, lambda i: (0,) * nd)


def _btile(shape, bm=_BM):
    nd = len(shape)
    return pl.BlockSpec((bm,) + shape[1:], lambda i: (i,) + (0,) * (nd - 1))


def kernel(x, mask, code_book, params):
    p = params
    grid = (_B // _BM,)
    big = 256
    gridb = (_B // big,)

    xf = jnp.reshape(x, (_B, _S0, _S0 * _D)).astype(_F32)
    mask = mask.astype(_F32)
    cb = code_book.astype(_F32)

    w1 = p['ew1'].reshape(128, 64)
    w2 = p['ew2'].reshape(256, 128)
    w3 = p['ew3'].reshape(512, 256)
    t1 = [p['dw1'][di].reshape(512, 128) for di in range(2)]
    t2 = [p['dw2'][di].reshape(256, 64) for di in range(2)]
    t3 = _band_convt(p['dw3'], 7)
    b3 = jnp.tile(p['eb3'], 5)[None, :]
    db3 = jnp.tile(p['db3'], 8)[None, :]
    cbt = cb.T
    cbn = jnp.sum(cb * cb, axis=1)[None, :]

    y1, st1 = _pcall(
        _enc1_body, grid=gridb,
        in_specs=[_btile((_B, 8, 256), big), _full((128, 64))],
        out_specs=[_btile((_B, 7, 448), big), _full((8, 448))],
        out_shape=[jax.ShapeDtypeStruct((_B, 7, 448), _F32),
                   jax.ShapeDtypeStruct((8, 448), _F32)],
        compiler_params=_CP,
    )(xf, w1)
    eg1, ebe1 = p['eg1'][None, :], p['ebe1'][None, :]

    y2, st2 = _pcall(
        _enc2_body, grid=gridb,
        in_specs=[_btile((_B, 7, 448), big), _full((8, 448)), _full((1, 64)),
                  _full((1, 64)), _full((256, 128))],
        out_specs=[_btile((_B, 6, 768), big), _full((8, 768))],
        out_shape=[jax.ShapeDtypeStruct((_B, 6, 768), _F32),
                   jax.ShapeDtypeStruct((8, 768), _F32)],
        compiler_params=_CP,
    )(y1, st1, eg1, ebe1, w2)

    ze, zq, g1, st3, vqs = _pcall(
        _vq_body, grid=grid,
        in_specs=[_btile((_B, 6, 768)), _full((8, 768)), _full((1, 128)),
                  _full((1, 128)), _full((512, 256)), _full((1, 1280)),
                  _full((256, 32)), _full((1, 32)), _full((32, 256)),
                  _full((512, 128)), _full((512, 128))],
        out_specs=[_btile((_B, 5, 1280)), _btile((_B, 5, 1280)),
                   _btile((_B, 6, 768)), _full((8, 768)), _full((8, 128))],
        out_shape=[jax.ShapeDtypeStruct((_B, 5, 1280), _F32),
                   jax.ShapeDtypeStruct((_B, 5, 1280), _F32),
                   jax.ShapeDtypeStruct((_B, 6, 768), _F32),
                   jax.ShapeDtypeStruct((8, 768), _F32),
                   jax.ShapeDtypeStruct((8, 128), _F32)],
        compiler_params=_CP,
    )(y2, st2, p['eg2'][None, :], p['ebe2'][None, :], w3, b3, cbt, cbn,
      cb, t1[0], t1[1])

    g2, st4 = _pcall(
        _dec2_body, grid=gridb,
        in_specs=[_btile((_B, 6, 768), big), _full((8, 768)), _full((1, 128)),
                  _full((1, 128)), _full((256, 64)), _full((256, 64))],
        out_specs=[_btile((_B, 7, 448), big), _full((8, 448))],
        out_shape=[jax.ShapeDtypeStruct((_B, 7, 448), _F32),
                   jax.ShapeDtypeStruct((8, 448), _F32)],
        compiler_params=_CP,
    )(g1, st3, p['dg1'][None, :], p['dbe1'][None, :], t2[0], t2[1])

    vx, vm, rec = _pcall(
        _dec3_body, grid=gridb,
        in_specs=[_btile((_B, 7, 448), big), _full((8, 448)), _full((1, 64)),
                  _full((1, 64)), _full((448, 256)), _full((448, 256)),
                  _full((1, 256)), _btile((_B, 8, 256), big),
                  _btile((_B, 64), big)],
        out_specs=[_btile((_B, 8, 256), big), _btile((_B, 32), big),
                   _full((8, 128))],
        out_shape=[jax.ShapeDtypeStruct((_B, 8, 256), _F32),
                   jax.ShapeDtypeStruct((_B, 32), _F32),
                   jax.ShapeDtypeStruct((8, 128), _F32)],
        compiler_params=_CP,
    )(g2, st4, p['dg2'][None, :], p['dbe2'][None, :], t3[0], t3[1], db3,
      xf, mask)

    ze_out = jnp.reshape(ze, (_B * 25, _CBD))
    zq_out = jnp.reshape(zq, (_B * 25, _CBD))
    vq_x = jnp.reshape(vx, (_B, _L, _D))
    recon = rec[0, 0] / float(_B * _L * _D)
    vq_term = vqs[0, 0] / float(_B * 25 * _CBD)
    loss = recon + vq_term + _BETA * vq_term
    return (vm, vq_x, ze_out, zq_out, loss)
